# Initial kernel scaffold; baseline (speedup 1.0000x reference)
import jax
import jax.numpy as jnp
from jax.experimental import pallas as pl


def kernel(feature, target, center_point):
    raise NotImplementedError("write your pallas kernel here")


# speedup vs baseline: 3.0027x; 3.0027x over previous
import functools

import jax
import jax.numpy as jnp
from jax import lax
from jax.experimental import pallas as pl
from jax.experimental.pallas import tpu as pltpu
from jax.experimental.pallas import tpu_sc as plsc

F = 128
C = 1000
CP = 1024
NC = 2
NS = 16
L = 16
R = 128


def _vec_sqrt(x):
    i = plsc.bitcast(x, jnp.int32)
    i = jnp.full((L,), 0x1FBD1DF5, jnp.int32) + lax.shift_right_logical(
        i, jnp.full((L,), 1, jnp.int32))
    y = plsc.bitcast(i, jnp.float32)
    half = jnp.full((L,), 0.5, jnp.float32)
    y = half * (y + x / y)
    y = half * (y + x / y)
    y = half * (y + x / y)
    return y


def _sc_body(batch, feat_hbm, tgt_hbm, cent_hbm, s_out, n_out,
             idx_v, feat_v, cent_v, dist_v, ones_v, zero_v, s_sh, n_sh, sem):
    rpw = batch // (NC * NS)
    nchunk = rpw // R
    cid = lax.axis_index("c")
    sid = lax.axis_index("s")
    base = (cid * NS + sid) * rpw

    for i in range(CP // L):
        zero_v[pl.ds(i * L, L)] = jnp.zeros((L,), jnp.float32)
    for i in range(R // L):
        ones_v[pl.ds(i * L, L)] = jnp.full((L,), 1.0, jnp.float32)

    @pl.when(sid == 0)
    def _():
        pltpu.sync_copy(zero_v, s_sh)
        pltpu.sync_copy(zero_v, n_sh)

    plsc.subcore_barrier()

    for ci in range(nchunk):
        row0 = base + ci * R
        pltpu.sync_copy(tgt_hbm.at[pl.ds(row0, R)], idx_v)
        pltpu.async_copy(cent_hbm.at[idx_v], cent_v, sem).wait()
        pltpu.sync_copy(feat_hbm.at[pl.ds(row0, R)], feat_v)

        lane = lax.iota(jnp.int32, L)

        @pl.loop(0, R // L)
        def _(g):
            totals = jnp.zeros((L,), jnp.float32)
            for ri in range(L):
                r = g * L + ri
                d0 = feat_v[r, pl.ds(0, L)] - cent_v[r, pl.ds(0, L)]
                acc = d0 * d0
                for k in range(1, F // L):
                    d = (feat_v[r, pl.ds(k * L, L)]
                         - cent_v[r, pl.ds(k * L, L)])
                    acc = acc + d * d
                t = lax.broadcast_in_dim(jnp.sum(acc), (L,), ())
                totals = jnp.where(lane == ri, t, totals)
            dist_v[pl.ds(pl.multiple_of(g * L, L), L)] = _vec_sqrt(totals)

        pltpu.sync_copy(dist_v, s_sh.at[idx_v], add=True)
        pltpu.sync_copy(ones_v, n_sh.at[idx_v], add=True)

    plsc.subcore_barrier()

    @pl.when(sid == 0)
    def _():
        pltpu.sync_copy(s_sh, s_out.at[cid])
        pltpu.sync_copy(n_sh, n_out.at[cid])


def _make_sc_kernel(batch):
    return pl.kernel(
        functools.partial(_sc_body, batch),
        out_type=(jax.ShapeDtypeStruct((NC, CP), jnp.float32),
                  jax.ShapeDtypeStruct((NC, CP), jnp.float32)),
        mesh=plsc.VectorSubcoreMesh(core_axis_name="c", subcore_axis_name="s",
                                    num_cores=NC, num_subcores=NS),
        compiler_params=pltpu.CompilerParams(needs_layout_passes=False),
        scratch_types=[
            pltpu.VMEM((R,), jnp.int32),
            pltpu.VMEM((R, F), jnp.float32),
            pltpu.VMEM((R, F), jnp.float32),
            pltpu.VMEM((R,), jnp.float32),
            pltpu.VMEM((R,), jnp.float32),
            pltpu.VMEM((CP,), jnp.float32),
            pltpu.VMEM_SHARED((CP,), jnp.float32),
            pltpu.VMEM_SHARED((CP,), jnp.float32),
            pltpu.SemaphoreType.DMA,
        ],
    )


def _epi_body(s_ref, n_ref, o_ref):
    s = s_ref[0:1, :] + s_ref[1:2, :]
    n = n_ref[0:1, :] + n_ref[1:2, :]
    safe_n = jnp.where(n > 0.0, n, 1.0)
    o_ref[0] = jnp.sum(jnp.where(n > 0.0, s / safe_n, 0.0))


_epilogue = pl.pallas_call(
    _epi_body,
    out_shape=jax.ShapeDtypeStruct((1,), jnp.float32),
    out_specs=pl.BlockSpec(memory_space=pltpu.SMEM),
)


def kernel(feature, target, center_point):
    batch = feature.shape[0]
    s, n = _make_sc_kernel(batch)(feature, target.astype(jnp.int32),
                                  center_point)
    return _epilogue(s, n)[0]


# speedup vs baseline: 4.0243x; 1.3403x over previous
import functools

import jax
import jax.numpy as jnp
from jax import lax
from jax.experimental import pallas as pl
from jax.experimental.pallas import tpu as pltpu
from jax.experimental.pallas import tpu_sc as plsc

F = 128
C = 1000
CP = 1024
NC = 2
NS = 16
L = 16
R = 128


def _vec_sqrt(x):
    i = plsc.bitcast(x, jnp.int32)
    i = jnp.full((L,), 0x1FBD1DF5, jnp.int32) + lax.shift_right_logical(
        i, jnp.full((L,), 1, jnp.int32))
    y = plsc.bitcast(i, jnp.float32)
    half = jnp.full((L,), 0.5, jnp.float32)
    y = half * (y + x / y)
    y = half * (y + x / y)
    y = half * (y + x / y)
    return y


def _sc_body(batch, feat_hbm, tgt_hbm, cent_hbm, s_out, n_out,
             idx0, idx1, feat0, feat1, cent0, cent1, dist_v, ones_v, zero_v,
             didx_v, accm_v, s_sh, n_sh, sem0, sem1):
    rpw = batch // (NC * NS)
    nchunk = rpw // R
    cid = lax.axis_index("c")
    sid = lax.axis_index("s")
    base = (cid * NS + sid) * rpw

    idx = (idx0, idx1)
    feat = (feat0, feat1)
    cent = (cent0, cent1)
    sem = (sem0, sem1)

    for i in range(CP // L):
        zero_v[pl.ds(i * L, L)] = jnp.zeros((L,), jnp.float32)
    for i in range(R // L):
        ones_v[pl.ds(i * L, L)] = jnp.full((L,), 1.0, jnp.float32)

    lane = lax.iota(jnp.int32, L)
    for d in range(L):
        t = lane + jnp.full((L,), d, jnp.int32)
        t = jnp.where(t >= L, t - jnp.full((L,), L, jnp.int32), t)
        didx_v[pl.ds(d * L, L)] = lane * jnp.full((L,), L, jnp.int32) + t

    @pl.when(sid == 0)
    def _():
        pltpu.sync_copy(zero_v, s_sh)
        pltpu.sync_copy(zero_v, n_sh)

    plsc.subcore_barrier()

    descs = [None, None]

    def prefetch(ci):
        p = ci % 2
        row0 = base + ci * R
        pltpu.sync_copy(tgt_hbm.at[pl.ds(row0, R)], idx[p])
        d_cent = pltpu.async_copy(cent_hbm.at[idx[p]], cent[p], sem[p])
        d_feat = pltpu.async_copy(feat_hbm.at[pl.ds(row0, R)], feat[p], sem[p])
        descs[p] = (d_cent, d_feat)

    prefetch(0)
    for ci in range(nchunk):
        p = ci % 2
        if ci + 1 < nchunk:
            prefetch(ci + 1)
        for d in descs[p]:
            d.wait()

        feat_v, cent_v = feat[p], cent[p]

        @pl.loop(0, R // L)
        def _(g):
            for ri in range(L):
                r = g * L + ri
                sq = []
                for k in range(F // L):
                    d = (feat_v[r, pl.ds(k * L, L)]
                         - cent_v[r, pl.ds(k * L, L)])
                    sq.append(d * d)
                while len(sq) > 1:
                    sq = [a + b for a, b in zip(sq[::2], sq[1::2])]
                accm_v[pl.ds(ri * L, L)] = sq[0]
            totals = jnp.zeros((L,), jnp.float32)
            for d in range(L):
                iv = didx_v[pl.ds(d * L, L)]
                totals = totals + plsc.load_gather(accm_v, [iv])
            dist_v[pl.ds(pl.multiple_of(g * L, L), L)] = _vec_sqrt(totals)

        pltpu.sync_copy(dist_v, s_sh.at[idx[p]], add=True)
        pltpu.sync_copy(ones_v, n_sh.at[idx[p]], add=True)

    plsc.subcore_barrier()

    @pl.when(sid == 0)
    def _():
        pltpu.sync_copy(s_sh, s_out.at[cid])
        pltpu.sync_copy(n_sh, n_out.at[cid])


def _make_sc_kernel(batch):
    return pl.kernel(
        functools.partial(_sc_body, batch),
        out_type=(jax.ShapeDtypeStruct((NC, CP), jnp.float32),
                  jax.ShapeDtypeStruct((NC, CP), jnp.float32)),
        mesh=plsc.VectorSubcoreMesh(core_axis_name="c", subcore_axis_name="s",
                                    num_cores=NC, num_subcores=NS),
        compiler_params=pltpu.CompilerParams(needs_layout_passes=False),
        scratch_types=[
            pltpu.VMEM((R,), jnp.int32),
            pltpu.VMEM((R,), jnp.int32),
            pltpu.VMEM((R, F), jnp.float32),
            pltpu.VMEM((R, F), jnp.float32),
            pltpu.VMEM((R, F), jnp.float32),
            pltpu.VMEM((R, F), jnp.float32),
            pltpu.VMEM((R,), jnp.float32),
            pltpu.VMEM((R,), jnp.float32),
            pltpu.VMEM((CP,), jnp.float32),
            pltpu.VMEM((L * L,), jnp.int32),
            pltpu.VMEM((L * L,), jnp.float32),
            pltpu.VMEM_SHARED((CP,), jnp.float32),
            pltpu.VMEM_SHARED((CP,), jnp.float32),
            pltpu.SemaphoreType.DMA,
            pltpu.SemaphoreType.DMA,
        ],
    )


def _epi_body(s_ref, n_ref, o_ref):
    s = s_ref[0:1, :] + s_ref[1:2, :]
    n = n_ref[0:1, :] + n_ref[1:2, :]
    safe_n = jnp.where(n > 0.0, n, 1.0)
    o_ref[0] = jnp.sum(jnp.where(n > 0.0, s / safe_n, 0.0))


_epilogue = pl.pallas_call(
    _epi_body,
    out_shape=jax.ShapeDtypeStruct((1,), jnp.float32),
    out_specs=pl.BlockSpec(memory_space=pltpu.SMEM),
)


def kernel(feature, target, center_point):
    batch = feature.shape[0]
    s, n = _make_sc_kernel(batch)(feature, target.astype(jnp.int32),
                                  center_point)
    return _epilogue(s, n)[0]
